# Initial kernel scaffold; baseline (speedup 1.0000x reference)
import jax
import jax.numpy as jnp
from jax.experimental import pallas as pl


def kernel(action, table):
    raise NotImplementedError("write your pallas kernel here")


# speedup vs baseline: 5.0313x; 5.0313x over previous
import functools

import jax
import jax.numpy as jnp
from jax import lax
from jax.experimental import pallas as pl
from jax.experimental.pallas import tpu as pltpu
from jax.experimental.pallas import tpu_sc as plsc

_BATCH = 16384
_HIST = 200
_EMBED = 32
_B = _BATCH * _HIST
_LANES = 128
_SUB = 8
_CHUNK = _SUB * _LANES
_NW = 32
_ROWS_PER_W = _B // _NW
_CHUNKS_PER_W = _ROWS_PER_W // _CHUNK
_IDXROWS_PER_W = _ROWS_PER_W // _LANES


def _body(idx_hbm, table_hbm, out_hbm, idx_v, rows_v,
          sem_i0, sem_i1, sem_g0, sem_g1, sem_o0, sem_o1):
    nc = plsc.get_sparse_core_info().num_cores
    wid = lax.axis_index("s") * nc + lax.axis_index("c")
    base_r = wid * _IDXROWS_PER_W
    base_e = wid * _ROWS_PER_W
    sem_i = (sem_i0, sem_i1)
    sem_g = (sem_g0, sem_g1)
    sem_o = (sem_o0, sem_o1)

    def start_idx(ch, slot):
        pltpu.async_copy(
            idx_hbm.at[pl.ds(base_r + ch * _SUB, _SUB)], idx_v.at[slot],
            sem_i[slot])

    def chunk(ch, slot, first):
        pltpu.make_async_copy(
            idx_hbm.at[pl.ds(0, _SUB)], idx_v.at[slot], sem_i[slot]).wait()
        if not first:
            pltpu.make_async_copy(
                rows_v.at[slot], out_hbm.at[pl.ds(0, _CHUNK)],
                sem_o[slot]).wait()
        for j in range(_SUB):
            pltpu.async_copy(
                table_hbm.at[idx_v.at[slot, j]],
                rows_v.at[slot, pl.ds(j * _LANES, _LANES)],
                sem_g[slot])
        pltpu.make_async_copy(
            table_hbm.at[pl.ds(0, _CHUNK)], rows_v.at[slot],
            sem_g[slot]).wait()
        pltpu.async_copy(
            rows_v.at[slot], out_hbm.at[pl.ds(base_e + ch * _CHUNK, _CHUNK)],
            sem_o[slot])
        @pl.when(ch + 2 < _CHUNKS_PER_W)
        def _():
            start_idx(ch + 2, slot)

    start_idx(0, 0)
    start_idx(1, 1)
    chunk(0, 0, first=True)
    chunk(1, 1, first=True)

    def loop_body(g, carry):
        chunk(2 * g, 0, first=False)
        chunk(2 * g + 1, 1, first=False)
        return carry

    lax.fori_loop(1, _CHUNKS_PER_W // 2, loop_body, 0)
    pltpu.make_async_copy(
        rows_v.at[0], out_hbm.at[pl.ds(0, _CHUNK)], sem_o[0]).wait()
    pltpu.make_async_copy(
        rows_v.at[1], out_hbm.at[pl.ds(0, _CHUNK)], sem_o[1]).wait()


@functools.partial(jax.jit, static_argnames=())
def kernel(action, table):
    idx2d = jnp.reshape(action.astype(jnp.int32), (_B // _LANES, _LANES))
    mesh = plsc.VectorSubcoreMesh(core_axis_name="c", subcore_axis_name="s")
    out = pl.kernel(
        _body,
        out_type=jax.ShapeDtypeStruct((_B, _EMBED), jnp.float32),
        mesh=mesh,
        scratch_types=[
            pltpu.VMEM((2, _SUB, _LANES), jnp.int32),
            pltpu.VMEM((2, _CHUNK, _EMBED), jnp.float32),
            pltpu.SemaphoreType.DMA,
            pltpu.SemaphoreType.DMA,
            pltpu.SemaphoreType.DMA,
            pltpu.SemaphoreType.DMA,
            pltpu.SemaphoreType.DMA,
            pltpu.SemaphoreType.DMA,
        ],
        compiler_params=pltpu.CompilerParams(use_tc_tiling_on_sc=False),
    )(idx2d, table)
    return jnp.reshape(out, (_BATCH, _HIST, _EMBED))


# speedup vs baseline: 5.0490x; 1.0035x over previous
import functools

import jax
import jax.numpy as jnp
from jax import lax
from jax.experimental import pallas as pl
from jax.experimental.pallas import tpu as pltpu
from jax.experimental.pallas import tpu_sc as plsc

_BATCH = 16384
_HIST = 200
_EMBED = 32
_B = _BATCH * _HIST
_LANES = 128
_SUB = 8
_CHUNK = _SUB * _LANES
_NW = 32
_ROWS_PER_W = _B // _NW
_CHUNKS_PER_W = _ROWS_PER_W // _CHUNK
_IDXROWS_PER_W = _ROWS_PER_W // _LANES


def _body(idx_hbm, table_hbm, out_hbm, idx_v, rows_v,
          sem_i0, sem_i1, sem_g0, sem_g1, sem_o0, sem_o1):
    nc = plsc.get_sparse_core_info().num_cores
    wid = lax.axis_index("s") * nc + lax.axis_index("c")
    base_r = wid * _IDXROWS_PER_W
    base_e = wid * _ROWS_PER_W
    sem_i = (sem_i0, sem_i1)
    sem_g = (sem_g0, sem_g1)
    sem_o = (sem_o0, sem_o1)

    def start_idx(ch, slot):
        pltpu.async_copy(
            idx_hbm.at[pl.ds(base_r + ch * _SUB, _SUB)], idx_v.at[slot],
            sem_i[slot])

    def wait_idx(slot):
        pltpu.make_async_copy(
            idx_hbm.at[pl.ds(0, _SUB)], idx_v.at[slot], sem_i[slot]).wait()

    def wait_store(slot):
        pltpu.make_async_copy(
            rows_v.at[slot], out_hbm.at[pl.ds(0, _CHUNK)], sem_o[slot]).wait()

    def fire(slot):
        for j in range(_SUB):
            pltpu.async_copy(
                table_hbm.at[idx_v.at[slot, j]],
                rows_v.at[slot, pl.ds(j * _LANES, _LANES)],
                sem_g[slot])

    def drain_store(ch, slot):
        pltpu.make_async_copy(
            table_hbm.at[pl.ds(0, _CHUNK)], rows_v.at[slot],
            sem_g[slot]).wait()
        pltpu.async_copy(
            rows_v.at[slot], out_hbm.at[pl.ds(base_e + ch * _CHUNK, _CHUNK)],
            sem_o[slot])

    def step(ch, slot):
        @pl.when(ch + 1 < _CHUNKS_PER_W)
        def _():
            wait_idx(slot ^ 1)

            @pl.when(ch >= 1)
            def _():
                wait_store(slot ^ 1)

            fire(slot ^ 1)

        drain_store(ch, slot)

        @pl.when(ch + 2 < _CHUNKS_PER_W)
        def _():
            start_idx(ch + 2, slot)

    start_idx(0, 0)
    start_idx(1, 1)
    wait_idx(0)
    fire(0)

    def loop_body(g, carry):
        step(2 * g, 0)
        step(2 * g + 1, 1)
        return carry

    lax.fori_loop(0, _CHUNKS_PER_W // 2, loop_body, 0)
    wait_store(0)
    wait_store(1)


@functools.partial(jax.jit, static_argnames=())
def kernel(action, table):
    idx2d = jnp.reshape(action.astype(jnp.int32), (_B // _LANES, _LANES))
    mesh = plsc.VectorSubcoreMesh(core_axis_name="c", subcore_axis_name="s")
    out = pl.kernel(
        _body,
        out_type=jax.ShapeDtypeStruct((_B, _EMBED), jnp.float32),
        mesh=mesh,
        scratch_types=[
            pltpu.VMEM((2, _SUB, _LANES), jnp.int32),
            pltpu.VMEM((2, _CHUNK, _EMBED), jnp.float32),
            pltpu.SemaphoreType.DMA,
            pltpu.SemaphoreType.DMA,
            pltpu.SemaphoreType.DMA,
            pltpu.SemaphoreType.DMA,
            pltpu.SemaphoreType.DMA,
            pltpu.SemaphoreType.DMA,
        ],
        compiler_params=pltpu.CompilerParams(use_tc_tiling_on_sc=False),
    )(idx2d, table)
    return jnp.reshape(out, (_BATCH, _HIST, _EMBED))
